# baseline (speedup 1.0000x reference)
import jax
import jax.numpy as jnp
from jax.experimental import pallas as pl
from jax.experimental.pallas import tpu as pltpu

N = 10000
D = 128
TM = 200
TM2 = 5000
S1 = N // TM
S2 = N // TM2
EPS = 1e-5


def _gin_kernel(h_full_ref, adj_ref, w1_ref, b1_ref, g1_ref, be1_ref,
                w2_ref, b2_ref, g2_ref, be2_ref, out_ref,
                z1_scratch, z2_scratch, stats_scratch, hb_scratch):
    t = pl.program_id(0)

    @pl.when(t < S1)
    def _phase_matmul():
        @pl.when(t == 0)
        def _cast_h():
            hb_scratch[...] = h_full_ref[...].astype(jnp.bfloat16)

        pooled = jnp.dot(adj_ref[...].astype(jnp.bfloat16),
                         hb_scratch[...],
                         preferred_element_type=jnp.float32)
        pooled = pooled + h_full_ref[pl.ds(t * TM, TM), :]
        z1 = jnp.dot(pooled, w1_ref[...],
                     preferred_element_type=jnp.float32) + b1_ref[...]
        z1_scratch[pl.ds(t * TM, TM), :] = z1

        @pl.when(t == 0)
        def _init():
            stats_scratch[...] = jnp.zeros_like(stats_scratch)

        zr = z1.reshape(TM // 8, 8, D)
        stats_scratch[0:8, :] += jnp.sum(zr, axis=0)
        stats_scratch[8:16, :] += jnp.sum(zr * zr, axis=0)

    @pl.when(jnp.logical_and(t >= S1, t < S1 + S2))
    def _phase_mlp():
        i = t - S1
        mean = jnp.sum(stats_scratch[0:8, :], axis=0, keepdims=True) * (1.0 / N)
        var = (jnp.sum(stats_scratch[8:16, :], axis=0, keepdims=True) * (1.0 / N)
               - mean * mean)
        scale = g1_ref[...] * jax.lax.rsqrt(var + EPS)
        shift = be1_ref[...] - mean * scale
        a = jnp.maximum(z1_scratch[pl.ds(i * TM2, TM2), :] * scale + shift,
                        0.0)
        z2 = jnp.dot(a, w2_ref[...],
                     preferred_element_type=jnp.float32) + b2_ref[...]
        z2_scratch[pl.ds(i * TM2, TM2), :] = z2

        z2r = z2.reshape(TM2 // 8, 8, D)
        stats_scratch[16:24, :] += jnp.sum(z2r, axis=0)
        stats_scratch[24:32, :] += jnp.sum(z2r * z2r, axis=0)

    @pl.when(t >= S1 + S2)
    def _phase_norm():
        i = t - (S1 + S2)
        mean = jnp.sum(stats_scratch[16:24, :], axis=0, keepdims=True) * (1.0 / N)
        var = (jnp.sum(stats_scratch[24:32, :], axis=0, keepdims=True) * (1.0 / N)
               - mean * mean)
        scale = g2_ref[...] * jax.lax.rsqrt(var + EPS)
        shift = be2_ref[...] - mean * scale
        z2 = z2_scratch[pl.ds(i * TM2, TM2), :]
        out_ref[...] = jnp.maximum(z2 * scale + shift, 0.0)


def kernel(h, Adj_block, padded_neighbor_list, W1, b1, bn1_gamma, bn1_beta,
           W2, b2, bn2_gamma, bn2_beta):
    del padded_neighbor_list
    b1r = b1.reshape(1, D)
    b2r = b2.reshape(1, D)
    g1 = bn1_gamma.reshape(1, D)
    be1 = bn1_beta.reshape(1, D)
    g2 = bn2_gamma.reshape(1, D)
    be2 = bn2_beta.reshape(1, D)
    const = lambda t: (0, 0)

    out = pl.pallas_call(
        _gin_kernel,
        grid=(S1 + 2 * S2,),
        in_specs=[
            pl.BlockSpec((N, D), const),
            pl.BlockSpec((TM, N), lambda t: (jnp.minimum(t, S1 - 1), 0)),
            pl.BlockSpec((D, D), const),
            pl.BlockSpec((1, D), const),
            pl.BlockSpec((1, D), const),
            pl.BlockSpec((1, D), const),
            pl.BlockSpec((D, D), const),
            pl.BlockSpec((1, D), const),
            pl.BlockSpec((1, D), const),
            pl.BlockSpec((1, D), const),
        ],
        out_specs=pl.BlockSpec(
            (TM2, D), lambda t: (jnp.maximum(t - (S1 + S2), 0), 0)),
        out_shape=jax.ShapeDtypeStruct((N, D), jnp.float32),
        scratch_shapes=[
            pltpu.VMEM((N, D), jnp.float32),
            pltpu.VMEM((N, D), jnp.float32),
            pltpu.VMEM((32, D), jnp.float32),
            pltpu.VMEM((N, D), jnp.bfloat16),
        ],
    )(h, Adj_block, W1, b1r, g1, be1, W2, b2r, g2, be2)

    return out


# speedup vs baseline: 1.0364x; 1.0364x over previous
import jax
import jax.numpy as jnp
from jax.experimental import pallas as pl
from jax.experimental.pallas import tpu as pltpu

N = 10000
D = 128
TM = 400
TM2 = 5000
S1 = N // TM
S2 = N // TM2
EPS = 1e-5


def _gin_kernel(h_full_ref, adj_ref, w1_ref, b1_ref, g1_ref, be1_ref,
                w2_ref, b2_ref, g2_ref, be2_ref, out_ref,
                z1_scratch, z2_scratch, stats_scratch, hb_scratch):
    t = pl.program_id(0)

    @pl.when(t < S1)
    def _phase_matmul():
        @pl.when(t == 0)
        def _cast_h():
            hb_scratch[...] = h_full_ref[...].astype(jnp.bfloat16)

        pooled = jnp.dot(adj_ref[...].astype(jnp.bfloat16),
                         hb_scratch[...],
                         preferred_element_type=jnp.float32)
        pooled = pooled + h_full_ref[pl.ds(t * TM, TM), :]
        z1 = jnp.dot(pooled, w1_ref[...],
                     preferred_element_type=jnp.float32) + b1_ref[...]
        z1_scratch[pl.ds(t * TM, TM), :] = z1

        @pl.when(t == 0)
        def _init():
            stats_scratch[...] = jnp.zeros_like(stats_scratch)

        zr = z1.reshape(TM // 8, 8, D)
        stats_scratch[0:8, :] += jnp.sum(zr, axis=0)
        stats_scratch[8:16, :] += jnp.sum(zr * zr, axis=0)

    @pl.when(jnp.logical_and(t >= S1, t < S1 + S2))
    def _phase_mlp():
        i = t - S1
        mean = jnp.sum(stats_scratch[0:8, :], axis=0, keepdims=True) * (1.0 / N)
        var = (jnp.sum(stats_scratch[8:16, :], axis=0, keepdims=True) * (1.0 / N)
               - mean * mean)
        scale = g1_ref[...] * jax.lax.rsqrt(var + EPS)
        shift = be1_ref[...] - mean * scale
        a = jnp.maximum(z1_scratch[pl.ds(i * TM2, TM2), :] * scale + shift,
                        0.0)
        z2 = jnp.dot(a, w2_ref[...],
                     preferred_element_type=jnp.float32) + b2_ref[...]
        z2_scratch[pl.ds(i * TM2, TM2), :] = z2

        z2r = z2.reshape(TM2 // 8, 8, D)
        stats_scratch[16:24, :] += jnp.sum(z2r, axis=0)
        stats_scratch[24:32, :] += jnp.sum(z2r * z2r, axis=0)

    @pl.when(t >= S1 + S2)
    def _phase_norm():
        i = t - (S1 + S2)
        mean = jnp.sum(stats_scratch[16:24, :], axis=0, keepdims=True) * (1.0 / N)
        var = (jnp.sum(stats_scratch[24:32, :], axis=0, keepdims=True) * (1.0 / N)
               - mean * mean)
        scale = g2_ref[...] * jax.lax.rsqrt(var + EPS)
        shift = be2_ref[...] - mean * scale
        z2 = z2_scratch[pl.ds(i * TM2, TM2), :]
        out_ref[...] = jnp.maximum(z2 * scale + shift, 0.0)


def kernel(h, Adj_block, padded_neighbor_list, W1, b1, bn1_gamma, bn1_beta,
           W2, b2, bn2_gamma, bn2_beta):
    del padded_neighbor_list
    b1r = b1.reshape(1, D)
    b2r = b2.reshape(1, D)
    g1 = bn1_gamma.reshape(1, D)
    be1 = bn1_beta.reshape(1, D)
    g2 = bn2_gamma.reshape(1, D)
    be2 = bn2_beta.reshape(1, D)
    const = lambda t: (0, 0)

    out = pl.pallas_call(
        _gin_kernel,
        grid=(S1 + 2 * S2,),
        in_specs=[
            pl.BlockSpec((N, D), const),
            pl.BlockSpec((TM, N), lambda t: (jnp.minimum(t, S1 - 1), 0)),
            pl.BlockSpec((D, D), const),
            pl.BlockSpec((1, D), const),
            pl.BlockSpec((1, D), const),
            pl.BlockSpec((1, D), const),
            pl.BlockSpec((D, D), const),
            pl.BlockSpec((1, D), const),
            pl.BlockSpec((1, D), const),
            pl.BlockSpec((1, D), const),
        ],
        out_specs=pl.BlockSpec(
            (TM2, D), lambda t: (jnp.maximum(t - (S1 + S2), 0), 0)),
        out_shape=jax.ShapeDtypeStruct((N, D), jnp.float32),
        scratch_shapes=[
            pltpu.VMEM((N, D), jnp.float32),
            pltpu.VMEM((N, D), jnp.float32),
            pltpu.VMEM((32, D), jnp.float32),
            pltpu.VMEM((N, D), jnp.bfloat16),
        ],
    )(h, Adj_block, W1, b1r, g1, be1, W2, b2r, g2, be2)

    return out


# speedup vs baseline: 1.0419x; 1.0053x over previous
import jax
import jax.numpy as jnp
from jax.experimental import pallas as pl
from jax.experimental.pallas import tpu as pltpu

N = 10000
D = 128
TM = 400
TM2 = 10000
S1 = N // TM
S2 = N // TM2
EPS = 1e-5


def _gin_kernel(h_full_ref, adj_ref, w1_ref, b1_ref, g1_ref, be1_ref,
                w2_ref, b2_ref, g2_ref, be2_ref, out_ref,
                z1_scratch, z2_scratch, stats_scratch, hb_scratch):
    t = pl.program_id(0)

    @pl.when(t < S1)
    def _phase_matmul():
        @pl.when(t == 0)
        def _cast_h():
            hb_scratch[...] = h_full_ref[...].astype(jnp.bfloat16)

        pooled = jnp.dot(adj_ref[...].astype(jnp.bfloat16),
                         hb_scratch[...],
                         preferred_element_type=jnp.float32)
        pooled = pooled + h_full_ref[pl.ds(t * TM, TM), :]
        z1 = jnp.dot(pooled, w1_ref[...],
                     preferred_element_type=jnp.float32) + b1_ref[...]
        z1_scratch[pl.ds(t * TM, TM), :] = z1

        @pl.when(t == 0)
        def _init():
            stats_scratch[...] = jnp.zeros_like(stats_scratch)

        zr = z1.reshape(TM // 8, 8, D)
        stats_scratch[0:8, :] += jnp.sum(zr, axis=0)
        stats_scratch[8:16, :] += jnp.sum(zr * zr, axis=0)

    @pl.when(jnp.logical_and(t >= S1, t < S1 + S2))
    def _phase_mlp():
        i = t - S1
        mean = jnp.sum(stats_scratch[0:8, :], axis=0, keepdims=True) * (1.0 / N)
        var = (jnp.sum(stats_scratch[8:16, :], axis=0, keepdims=True) * (1.0 / N)
               - mean * mean)
        scale = g1_ref[...] * jax.lax.rsqrt(var + EPS)
        shift = be1_ref[...] - mean * scale
        a = jnp.maximum(z1_scratch[pl.ds(i * TM2, TM2), :] * scale + shift,
                        0.0)
        z2 = jnp.dot(a, w2_ref[...],
                     preferred_element_type=jnp.float32) + b2_ref[...]
        z2_scratch[pl.ds(i * TM2, TM2), :] = z2

        z2r = z2.reshape(TM2 // 8, 8, D)
        stats_scratch[16:24, :] += jnp.sum(z2r, axis=0)
        stats_scratch[24:32, :] += jnp.sum(z2r * z2r, axis=0)

    @pl.when(t >= S1 + S2)
    def _phase_norm():
        i = t - (S1 + S2)
        mean = jnp.sum(stats_scratch[16:24, :], axis=0, keepdims=True) * (1.0 / N)
        var = (jnp.sum(stats_scratch[24:32, :], axis=0, keepdims=True) * (1.0 / N)
               - mean * mean)
        scale = g2_ref[...] * jax.lax.rsqrt(var + EPS)
        shift = be2_ref[...] - mean * scale
        z2 = z2_scratch[pl.ds(i * TM2, TM2), :]
        out_ref[...] = jnp.maximum(z2 * scale + shift, 0.0)


def kernel(h, Adj_block, padded_neighbor_list, W1, b1, bn1_gamma, bn1_beta,
           W2, b2, bn2_gamma, bn2_beta):
    del padded_neighbor_list
    b1r = b1.reshape(1, D)
    b2r = b2.reshape(1, D)
    g1 = bn1_gamma.reshape(1, D)
    be1 = bn1_beta.reshape(1, D)
    g2 = bn2_gamma.reshape(1, D)
    be2 = bn2_beta.reshape(1, D)
    const = lambda t: (0, 0)

    out = pl.pallas_call(
        _gin_kernel,
        grid=(S1 + 2 * S2,),
        in_specs=[
            pl.BlockSpec((N, D), const),
            pl.BlockSpec((TM, N), lambda t: (jnp.minimum(t, S1 - 1), 0)),
            pl.BlockSpec((D, D), const),
            pl.BlockSpec((1, D), const),
            pl.BlockSpec((1, D), const),
            pl.BlockSpec((1, D), const),
            pl.BlockSpec((D, D), const),
            pl.BlockSpec((1, D), const),
            pl.BlockSpec((1, D), const),
            pl.BlockSpec((1, D), const),
        ],
        out_specs=pl.BlockSpec(
            (TM2, D), lambda t: (jnp.maximum(t - (S1 + S2), 0), 0)),
        out_shape=jax.ShapeDtypeStruct((N, D), jnp.float32),
        scratch_shapes=[
            pltpu.VMEM((N, D), jnp.float32),
            pltpu.VMEM((N, D), jnp.float32),
            pltpu.VMEM((32, D), jnp.float32),
            pltpu.VMEM((N, D), jnp.bfloat16),
        ],
    )(h, Adj_block, W1, b1r, g1, be1, W2, b2r, g2, be2)

    return out
